# baseline (speedup 1.0000x reference)
import functools

import jax
import jax.numpy as jnp
from jax.experimental import pallas as pl
from jax.experimental.pallas import tpu as pltpu

_B = 64
_P = 32
_HD = 64
_ED = 64
_D1 = 512
_D2 = 1024
_PK = 8
_S = 32
_JC = 8


def _decoder_kernel(pos_ref, h_ref, ms_ref, hs_ref, cu_ref, w2_ref,
                    b2_ref, out_ref):
    pos = pos_ref[...].reshape(_S * _P, _PK)
    h = h_ref[...].reshape(_S * _P, _HD)
    dn = (((1,), (0,)), ((), ()))
    v = jax.lax.dot_general(pos, ms_ref[...], dn,
                            preferred_element_type=jnp.float32)
    ht = jax.lax.dot_general(h, hs_ref[...], dn,
                             preferred_element_type=jnp.float32)
    u = v + ht + cu_ref[0][None, :]
    v4 = v.reshape(_S, _P, _D1).astype(jnp.bfloat16)
    u4 = u.reshape(_S, _P, _D1).astype(jnp.bfloat16)
    m = None
    for jc in range(_P // _JC):
        uc = u4[:, jc * _JC:(jc + 1) * _JC, :]
        z1c = jnp.maximum(uc[:, :, None, :] - v4[:, None, :, :],
                          jnp.bfloat16(0.0))
        yc = jax.lax.dot_general(z1c.reshape(_S * _JC * _P, _D1), w2_ref[...],
                                 dn, preferred_element_type=jnp.float32)
        mc = jnp.max(yc.reshape(_S, _JC, _P, _D2), axis=1)
        m = mc if m is None else jnp.maximum(m, mc)
    m32 = m.reshape(_S * _P, _D2)
    out_ref[...] = jnp.maximum(m32 + b2_ref[0][None, :], 0.0)


@functools.partial(jax.jit, static_argnames=())
def kernel(h_states, seq_start_end, end_pos, W_sp, b_sp, W1, b1, g1, be1,
           rm1, rv1, W2, b2, g2, be2, rm2, rv2):
    del seq_start_end
    f32 = jnp.float32
    s1 = g1 / jnp.sqrt(rv1 + 1e-5)
    W1e = W1[:, :_ED]
    W1h = W1[:, _ED:]
    Ms = (W_sp.T @ W1e.T) * s1[None, :]
    Ms_pad = jnp.zeros((_PK, _D1), f32).at[:2, :].set(Ms)
    Hs = W1h.T * s1[None, :]
    cU = (b_sp @ W1e.T + b1) * s1 + (be1 - rm1 * s1)
    s2 = g2 / jnp.sqrt(rv2 + 1e-5)
    W2f = (W2 * s2[:, None]).T.astype(jnp.bfloat16)
    b2f = b2 * s2 + (be2 - rm2 * s2)

    pos = end_pos.reshape(_B, _P, 2)
    pos_pad = jnp.zeros((_B, _P, _PK), f32).at[:, :, :2].set(pos)
    h3 = h_states.reshape(_B, _P, _HD)

    out = pl.pallas_call(
        _decoder_kernel,
        grid=(_B // _S,),
        in_specs=[
            pl.BlockSpec((_S, _P, _PK), lambda b: (b, 0, 0)),
            pl.BlockSpec((_S, _P, _HD), lambda b: (b, 0, 0)),
            pl.BlockSpec((_PK, _D1), lambda b: (0, 0)),
            pl.BlockSpec((_HD, _D1), lambda b: (0, 0)),
            pl.BlockSpec((1, _D1), lambda b: (0, 0)),
            pl.BlockSpec((_D1, _D2), lambda b: (0, 0)),
            pl.BlockSpec((1, _D2), lambda b: (0, 0)),
        ],
        out_specs=pl.BlockSpec((_S * _P, _D2), lambda b: (b, 0)),
        out_shape=jax.ShapeDtypeStruct((_B * _P, _D2), f32),
        compiler_params=pltpu.CompilerParams(
            dimension_semantics=("arbitrary",),
        ),
    )(pos_pad, h3, Ms_pad, Hs, cU.reshape(1, _D1), W2f, b2f.reshape(1, _D2))
    return out


# speedup vs baseline: 1.2688x; 1.2688x over previous
import functools

import jax
import jax.numpy as jnp
from jax.experimental import pallas as pl
from jax.experimental.pallas import tpu as pltpu

_B = 64
_P = 32
_HD = 64
_ED = 64
_D1 = 512
_D2 = 1024
_PK = 8
_S = 16
_JC = 16


def _decoder_kernel(pos_ref, h_ref, ms_ref, hs_ref, cu_ref, w2_ref,
                    b2_ref, out_ref):
    pos = pos_ref[...].reshape(_S * _P, _PK)
    h = h_ref[...].reshape(_S * _P, _HD)
    dn = (((1,), (0,)), ((), ()))
    v = jax.lax.dot_general(pos, ms_ref[...], dn,
                            preferred_element_type=jnp.float32)
    ht = jax.lax.dot_general(h, hs_ref[...], dn,
                             preferred_element_type=jnp.float32)
    u = v + ht + cu_ref[0][None, :]
    v4 = v.reshape(_S, _P, _D1).astype(jnp.bfloat16)
    u4 = u.reshape(_S, _P, _D1).astype(jnp.bfloat16)
    m = None
    for jc in range(_P // _JC):
        uc = u4[:, jc * _JC:(jc + 1) * _JC, :]
        z1c = jnp.maximum(uc[:, :, None, :] - v4[:, None, :, :],
                          jnp.bfloat16(0.0))
        yc = jax.lax.dot_general(z1c.reshape(_S * _JC * _P, _D1), w2_ref[...],
                                 dn, preferred_element_type=jnp.float32)
        mc = jnp.max(yc.reshape(_S, _JC, _P, _D2), axis=1)
        m = mc if m is None else jnp.maximum(m, mc)
    m32 = m.reshape(_S * _P, _D2)
    out_ref[...] = jnp.maximum(m32 + b2_ref[0][None, :], 0.0)


@functools.partial(jax.jit, static_argnames=())
def kernel(h_states, seq_start_end, end_pos, W_sp, b_sp, W1, b1, g1, be1,
           rm1, rv1, W2, b2, g2, be2, rm2, rv2):
    del seq_start_end
    f32 = jnp.float32
    s1 = g1 / jnp.sqrt(rv1 + 1e-5)
    W1e = W1[:, :_ED]
    W1h = W1[:, _ED:]
    Ms = (W_sp.T @ W1e.T) * s1[None, :]
    Ms_pad = jnp.zeros((_PK, _D1), f32).at[:2, :].set(Ms)
    Hs = W1h.T * s1[None, :]
    cU = (b_sp @ W1e.T + b1) * s1 + (be1 - rm1 * s1)
    s2 = g2 / jnp.sqrt(rv2 + 1e-5)
    W2f = (W2 * s2[:, None]).T.astype(jnp.bfloat16)
    b2f = b2 * s2 + (be2 - rm2 * s2)

    pos = end_pos.reshape(_B, _P, 2)
    pos_pad = jnp.zeros((_B, _P, _PK), f32).at[:, :, :2].set(pos)
    h3 = h_states.reshape(_B, _P, _HD)

    out = pl.pallas_call(
        _decoder_kernel,
        grid=(_B // _S,),
        in_specs=[
            pl.BlockSpec((_S, _P, _PK), lambda b: (b, 0, 0)),
            pl.BlockSpec((_S, _P, _HD), lambda b: (b, 0, 0)),
            pl.BlockSpec((_PK, _D1), lambda b: (0, 0)),
            pl.BlockSpec((_HD, _D1), lambda b: (0, 0)),
            pl.BlockSpec((1, _D1), lambda b: (0, 0)),
            pl.BlockSpec((_D1, _D2), lambda b: (0, 0)),
            pl.BlockSpec((1, _D2), lambda b: (0, 0)),
        ],
        out_specs=pl.BlockSpec((_S * _P, _D2), lambda b: (b, 0)),
        out_shape=jax.ShapeDtypeStruct((_B * _P, _D2), f32),
        compiler_params=pltpu.CompilerParams(
            dimension_semantics=("arbitrary",),
        ),
    )(pos_pad, h3, Ms_pad, Hs, cU.reshape(1, _D1), W2f, b2f.reshape(1, _D2))
    return out


# speedup vs baseline: 1.2693x; 1.0004x over previous
import functools

import jax
import jax.numpy as jnp
from jax.experimental import pallas as pl
from jax.experimental.pallas import tpu as pltpu

_B = 64
_P = 32
_HD = 64
_ED = 64
_D1 = 512
_D2 = 1024
_PK = 8
_S = 16
_JC = 16


def _decoder_kernel(pos_ref, h_ref, ms_ref, hs_ref, cu_ref, w2_ref,
                    b2_ref, out_ref):
    pos = pos_ref[...].reshape(_S * _P, _PK)
    h = h_ref[...].reshape(_S * _P, _HD)
    dn = (((1,), (0,)), ((), ()))
    v = jax.lax.dot_general(pos, ms_ref[...], dn,
                            preferred_element_type=jnp.float32)
    ht = jax.lax.dot_general(h, hs_ref[...], dn,
                             preferred_element_type=jnp.float32)
    u = v + ht + cu_ref[0][None, :]
    v4 = v.reshape(_S, _P, _D1).astype(jnp.bfloat16)
    u4 = u.reshape(_S, _P, _D1).astype(jnp.bfloat16)
    m = None
    for jc in range(_P // _JC):
        uc = u4[:, jc * _JC:(jc + 1) * _JC, :]
        z1c = jnp.maximum(uc[:, :, None, :] - v4[:, None, :, :],
                          jnp.bfloat16(0.0))
        yc = jax.lax.dot_general(z1c.reshape(_S * _JC * _P, _D1), w2_ref[...],
                                 dn, preferred_element_type=jnp.float32)
        yb = yc.astype(jnp.bfloat16)
        mc = jnp.max(yb.reshape(_S, _JC, _P, _D2), axis=1)
        m = mc if m is None else jnp.maximum(m, mc)
    m32 = m.reshape(_S * _P, _D2).astype(jnp.float32)
    out_ref[...] = jnp.maximum(m32 + b2_ref[0][None, :], 0.0)


@functools.partial(jax.jit, static_argnames=())
def kernel(h_states, seq_start_end, end_pos, W_sp, b_sp, W1, b1, g1, be1,
           rm1, rv1, W2, b2, g2, be2, rm2, rv2):
    del seq_start_end
    f32 = jnp.float32
    s1 = g1 / jnp.sqrt(rv1 + 1e-5)
    W1e = W1[:, :_ED]
    W1h = W1[:, _ED:]
    Ms = (W_sp.T @ W1e.T) * s1[None, :]
    Ms_pad = jnp.zeros((_PK, _D1), f32).at[:2, :].set(Ms)
    Hs = W1h.T * s1[None, :]
    cU = (b_sp @ W1e.T + b1) * s1 + (be1 - rm1 * s1)
    s2 = g2 / jnp.sqrt(rv2 + 1e-5)
    W2f = (W2 * s2[:, None]).T.astype(jnp.bfloat16)
    b2f = b2 * s2 + (be2 - rm2 * s2)

    pos = end_pos.reshape(_B, _P, 2)
    pos_pad = jnp.zeros((_B, _P, _PK), f32).at[:, :, :2].set(pos)
    h3 = h_states.reshape(_B, _P, _HD)

    out = pl.pallas_call(
        _decoder_kernel,
        grid=(_B // _S,),
        in_specs=[
            pl.BlockSpec((_S, _P, _PK), lambda b: (b, 0, 0)),
            pl.BlockSpec((_S, _P, _HD), lambda b: (b, 0, 0)),
            pl.BlockSpec((_PK, _D1), lambda b: (0, 0)),
            pl.BlockSpec((_HD, _D1), lambda b: (0, 0)),
            pl.BlockSpec((1, _D1), lambda b: (0, 0)),
            pl.BlockSpec((_D1, _D2), lambda b: (0, 0)),
            pl.BlockSpec((1, _D2), lambda b: (0, 0)),
        ],
        out_specs=pl.BlockSpec((_S * _P, _D2), lambda b: (b, 0)),
        out_shape=jax.ShapeDtypeStruct((_B * _P, _D2), f32),
        compiler_params=pltpu.CompilerParams(
            dimension_semantics=("arbitrary",),
        ),
    )(pos_pad, h3, Ms_pad, Hs, cU.reshape(1, _D1), W2f, b2f.reshape(1, _D2))
    return out


# speedup vs baseline: 1.2716x; 1.0018x over previous
import functools

import jax
import jax.numpy as jnp
from jax.experimental import pallas as pl
from jax.experimental.pallas import tpu as pltpu

_B = 64
_P = 32
_HD = 64
_ED = 64
_D1 = 512
_D2 = 1024
_PK = 8
_S = 16
_JC = 16


def _decoder_kernel(pos_ref, h_ref, ms_ref, hs_ref, cu_ref, w2_ref,
                    b2_ref, out_ref):
    pos = pos_ref[...].reshape(_S * _P, _PK)
    h = h_ref[...].reshape(_S * _P, _HD)
    dn = (((1,), (0,)), ((), ()))
    v = jax.lax.dot_general(pos, ms_ref[...], dn,
                            preferred_element_type=jnp.float32)
    ht = jax.lax.dot_general(h, hs_ref[...], dn,
                             preferred_element_type=jnp.float32)
    u = v + ht + cu_ref[0][None, :]
    v4 = v.reshape(_S, _P, _D1).astype(jnp.bfloat16)
    u4 = u.reshape(_S, _P, _D1).astype(jnp.bfloat16)
    m = None
    for jc in range(_P // _JC):
        uc = u4[:, jc * _JC:(jc + 1) * _JC, :]
        z1c = jnp.maximum(uc[:, :, None, :] - v4[:, None, :, :],
                          jnp.bfloat16(0.0))
        yc = jax.lax.dot_general(z1c.reshape(_S * _JC * _P, _D1), w2_ref[...],
                                 dn, preferred_element_type=jnp.float32)
        yb = yc.astype(jnp.bfloat16)
        mc = jnp.max(yb.reshape(_S, _JC, _P, _D2), axis=1)
        m = mc if m is None else jnp.maximum(m, mc)
    m32 = m.reshape(_S * _P, _D2).astype(jnp.float32)
    out_ref[...] = jnp.maximum(m32 + b2_ref[0][None, :], 0.0)


@functools.partial(jax.jit, static_argnames=())
def kernel(h_states, seq_start_end, end_pos, W_sp, b_sp, W1, b1, g1, be1,
           rm1, rv1, W2, b2, g2, be2, rm2, rv2):
    del seq_start_end
    f32 = jnp.float32
    s1 = g1 / jnp.sqrt(rv1 + 1e-5)
    W1e = W1[:, :_ED]
    W1h = W1[:, _ED:]
    Ms = (W_sp.T @ W1e.T) * s1[None, :]
    Ms_pad = jnp.zeros((_PK, _D1), f32).at[:2, :].set(Ms)
    Hs = W1h.T * s1[None, :]
    cU = (b_sp @ W1e.T + b1) * s1 + (be1 - rm1 * s1)
    s2 = g2 / jnp.sqrt(rv2 + 1e-5)
    W2f = (W2 * s2[:, None]).T.astype(jnp.bfloat16)
    b2f = b2 * s2 + (be2 - rm2 * s2)

    pos = end_pos.reshape(_B, _P, 2)
    pos_pad = jnp.zeros((_B, _P, _PK), f32).at[:, :, :2].set(pos)
    h3 = h_states.reshape(_B, _P, _HD)

    out = pl.pallas_call(
        _decoder_kernel,
        grid=(_B // _S,),
        in_specs=[
            pl.BlockSpec((_S, _P, _PK), lambda b: (b, 0, 0)),
            pl.BlockSpec((_S, _P, _HD), lambda b: (b, 0, 0)),
            pl.BlockSpec((_PK, _D1), lambda b: (0, 0)),
            pl.BlockSpec((_HD, _D1), lambda b: (0, 0)),
            pl.BlockSpec((1, _D1), lambda b: (0, 0)),
            pl.BlockSpec((_D1, _D2), lambda b: (0, 0)),
            pl.BlockSpec((1, _D2), lambda b: (0, 0)),
        ],
        out_specs=pl.BlockSpec((_S * _P, _D2), lambda b: (b, 0)),
        out_shape=jax.ShapeDtypeStruct((_B * _P, _D2), f32),
        compiler_params=pltpu.CompilerParams(
            dimension_semantics=("arbitrary",),
            vmem_limit_bytes=100 * 1024 * 1024,
        ),
    )(pos_pad, h3, Ms_pad, Hs, cU.reshape(1, _D1), W2f, b2f.reshape(1, _D2))
    return out
